# Initial kernel scaffold; baseline (speedup 1.0000x reference)
import jax
import jax.numpy as jnp
from jax.experimental import pallas as pl


def kernel(q, k, v, mask, mem_keys, mem_values, scale_param):
    raise NotImplementedError("write your pallas kernel here")


# speedup vs baseline: 25.4385x; 25.4385x over previous
import functools
import math

import jax
import jax.numpy as jnp
from jax.experimental import pallas as pl
from jax.experimental.pallas import tpu as pltpu

NEG = -1e30
K_RETR = 32


def _norm_body(x_ref, o_ref):
    x = x_ref[...]
    n = jnp.sqrt(jnp.sum(x * x, axis=-1, keepdims=True))
    o_ref[...] = x / jnp.maximum(n, 1e-12)


def _l2norm_rows(x):
    return pl.pallas_call(
        _norm_body,
        out_shape=jax.ShapeDtypeStruct(x.shape, x.dtype),
    )(x)


def _attn_body(scale_ref, q_ref, kn_ref, v_ref, mask_ref, mk_ref, mv_ref,
               o_ref):
    h = pl.program_id(0)
    qb = q_ref[0]
    qn = qb / jnp.maximum(
        jnp.sqrt(jnp.sum(qb * qb, axis=-1, keepdims=True)), 1e-12)
    qn16 = qn.astype(jnp.bfloat16)
    scale = jnp.exp(scale_ref[h])

    kn = kn_ref[...].astype(jnp.bfloat16)
    mk = mk_ref[...].astype(jnp.bfloat16)

    sim_loc = jax.lax.dot_general(
        qn16, kn, (((1,), (1,)), ((), ())),
        preferred_element_type=jnp.float32) * scale
    mask_term = (-3.4e38) * (1.0 - mask_ref[0])
    sim_loc = sim_loc + mask_term[None, :]

    srch = jax.lax.dot_general(
        qn16, mk, (((1,), (1,)), ((), ())),
        preferred_element_type=jnp.float32) * scale

    m_mem = jnp.max(srch, axis=-1, keepdims=True)
    t = m_mem
    for _ in range(K_RETR - 1):
        t = jnp.max(jnp.where(srch < t, srch, NEG), axis=-1, keepdims=True)

    m_loc = jnp.max(sim_loc, axis=-1, keepdims=True)
    m = jnp.maximum(m_mem, m_loc)
    p_mem = jnp.where(srch >= t, jnp.exp(srch - m), 0.0)
    p_loc = jnp.exp(sim_loc - m)
    z = (jnp.sum(p_mem, axis=-1, keepdims=True)
         + jnp.sum(p_loc, axis=-1, keepdims=True))
    acc = (jax.lax.dot_general(p_loc.astype(jnp.bfloat16),
                               v_ref[...].astype(jnp.bfloat16),
                               (((1,), (0,)), ((), ())),
                               preferred_element_type=jnp.float32)
           + jax.lax.dot_general(p_mem.astype(jnp.bfloat16),
                                 mv_ref[...].astype(jnp.bfloat16),
                                 (((1,), (0,)), ((), ())),
                                 preferred_element_type=jnp.float32))
    o_ref[0] = acc / z


def kernel(q, k, v, mask, mem_keys, mem_values, scale_param):
    B, H, S, D = q.shape
    M = mem_keys.shape[1]
    assert B == 1
    SB = min(256, S)

    kn = _l2norm_rows(k[0])
    mkn = _l2norm_rows(mem_keys[0])
    scale = scale_param.reshape(H)

    out = pl.pallas_call(
        _attn_body,
        grid=(H, S // SB),
        in_specs=[
            pl.BlockSpec(memory_space=pltpu.SMEM),
            pl.BlockSpec((1, SB, D), lambda h, i: (h, i, 0)),
            pl.BlockSpec((S, D), lambda h, i: (0, 0)),
            pl.BlockSpec((S, D), lambda h, i: (0, 0)),
            pl.BlockSpec((1, S), lambda h, i: (0, 0)),
            pl.BlockSpec((M, D), lambda h, i: (0, 0)),
            pl.BlockSpec((M, D), lambda h, i: (0, 0)),
        ],
        out_specs=pl.BlockSpec((1, SB, D), lambda h, i: (h, i, 0)),
        out_shape=jax.ShapeDtypeStruct((H, S, D), jnp.float32),
    )(scale, q[0], kn, v[0], mask, mkn, mem_values[0])
    return out[None]
